# baseline (speedup 1.0000x reference)
import functools

import jax
import jax.numpy as jnp
from jax import lax
from jax.experimental import pallas as pl
from jax.experimental.pallas import tpu as pltpu
from jax.experimental.pallas import tpu_sc as plsc

_D = 64
_W = _D // 4
_NW = 32
_CHUNK = 1024


def _sc_gather_body(idx_hbm, w_hbm, s_hbm, rows_out, sc_out,
                    idx_v, rows_v, sc_v, sem, *, n_per_w):
    wid = lax.axis_index("s") * 2 + lax.axis_index("c")
    for k in range(n_per_w // _CHUNK):
        base = wid * n_per_w + k * _CHUNK
        pltpu.sync_copy(idx_hbm.at[pl.ds(base, _CHUNK)], idx_v)
        rows_dma = pltpu.async_copy(w_hbm.at[idx_v], rows_v, sem)
        sc_dma = pltpu.async_copy(s_hbm.at[idx_v], sc_v, sem)
        rows_dma.wait()
        sc_dma.wait()
        pltpu.sync_copy(rows_v, rows_out.at[pl.ds(base, _CHUNK)])
        pltpu.sync_copy(sc_v, sc_out.at[pl.ds(base, _CHUNK)])


@functools.cache
def _sc_gather(n):
    n_per_w = n // _NW
    mesh = plsc.VectorSubcoreMesh(core_axis_name="c", subcore_axis_name="s")
    return pl.kernel(
        functools.partial(_sc_gather_body, n_per_w=n_per_w),
        out_type=[
            jax.ShapeDtypeStruct((n, _W), jnp.int32),
            jax.ShapeDtypeStruct((n,), jnp.float32),
        ],
        mesh=mesh,
        compiler_params=pltpu.CompilerParams(use_tc_tiling_on_sc=False),
        scratch_types=[
            pltpu.VMEM((_CHUNK,), jnp.int32),
            pltpu.VMEM((_CHUNK, _W), jnp.int32),
            pltpu.VMEM((_CHUNK,), jnp.float32),
            pltpu.SemaphoreType.DMA,
        ],
    )


def _dequant_body(rows_ref, sc_ref, out_ref):
    w = rows_ref[...]
    s = sc_ref[...]
    planes = []
    for k in range(4):
        b = (w << (24 - 8 * k)) >> 24
        planes.append(b.astype(jnp.float32) * s)
    blk = w.shape[0]
    out_ref[...] = jnp.stack(planes, axis=-1).reshape(blk, _D)


@functools.cache
def _dequant(n, blk):
    return pl.pallas_call(
        _dequant_body,
        out_shape=jax.ShapeDtypeStruct((n, _D), jnp.float32),
        grid=(n // blk,),
        in_specs=[
            pl.BlockSpec((blk, _W), lambda i: (i, 0)),
            pl.BlockSpec((blk, 1), lambda i: (i, 0)),
        ],
        out_specs=pl.BlockSpec((blk, _D), lambda i: (i, 0)),
    )


def kernel(x, weight, scales):
    b, f = x.shape
    n = b * f
    nv = weight.shape[0]
    idx = x.reshape(n).astype(jnp.int32)
    w32 = lax.bitcast_convert_type(weight.reshape(nv, _W, 4), jnp.int32)
    scales_f32 = scales.astype(jnp.float32)
    rows_g, sc_g = _sc_gather(n)(idx, w32, scales_f32)
    out = _dequant(n, 1024)(rows_g, sc_g.reshape(n, 1))
    return out.astype(jnp.float16).reshape(b, f, _D)


# speedup vs baseline: 3.9616x; 3.9616x over previous
import functools

import jax
import jax.numpy as jnp
from jax import lax
from jax.experimental import pallas as pl
from jax.experimental.pallas import tpu as pltpu
from jax.experimental.pallas import tpu_sc as plsc

_D = 64
_W = _D // 4
_NW = 32
_CHUNK = 1024


def _sc_gather_body(idx_hbm, w_hbm, s_hbm, rows_out, sc_out,
                    idx_v, rows_v, sc_v, sem, *, n_per_w):
    wid = lax.axis_index("s") * 2 + lax.axis_index("c")
    for k in range(n_per_w // _CHUNK):
        base = wid * n_per_w + k * _CHUNK
        pltpu.sync_copy(idx_hbm.at[pl.ds(base, _CHUNK)], idx_v)
        rows_dma = pltpu.async_copy(w_hbm.at[idx_v], rows_v, sem)
        sc_dma = pltpu.async_copy(s_hbm.at[idx_v], sc_v, sem)
        rows_dma.wait()
        sc_dma.wait()
        pltpu.sync_copy(rows_v, rows_out.at[pl.ds(base, _CHUNK)])
        pltpu.sync_copy(sc_v, sc_out.at[pl.ds(base, _CHUNK)])


@functools.cache
def _sc_gather(n):
    n_per_w = n // _NW
    mesh = plsc.VectorSubcoreMesh(core_axis_name="c", subcore_axis_name="s")
    return pl.kernel(
        functools.partial(_sc_gather_body, n_per_w=n_per_w),
        out_type=[
            jax.ShapeDtypeStruct((n, _W), jnp.int32),
            jax.ShapeDtypeStruct((n,), jnp.float32),
        ],
        mesh=mesh,
        compiler_params=pltpu.CompilerParams(use_tc_tiling_on_sc=False),
        scratch_types=[
            pltpu.VMEM((_CHUNK,), jnp.int32),
            pltpu.VMEM((_CHUNK, _W), jnp.int32),
            pltpu.VMEM((_CHUNK,), jnp.float32),
            pltpu.SemaphoreType.DMA,
        ],
    )


def _dequant_body(rows_ref, sc_ref, out_ref):
    w = rows_ref[...]
    s = sc_ref[...]
    blk = w.shape[0]
    lane = lax.broadcasted_iota(jnp.int32, (blk, 512), 1)
    wrep = jnp.take_along_axis(w, lane // 4, axis=1)
    srep = jnp.take_along_axis(s, lane // _D, axis=1)
    k = lax.broadcasted_iota(jnp.int32, (blk, 512), 1) % 4
    b = (wrep << (24 - 8 * k)) >> 24
    out_ref[...] = b.astype(jnp.float32) * srep


@functools.cache
def _dequant(n, blk):
    return pl.pallas_call(
        _dequant_body,
        out_shape=jax.ShapeDtypeStruct((n // 8, 512), jnp.float32),
        grid=(n // 8 // blk,),
        in_specs=[
            pl.BlockSpec((blk, 128), lambda i: (i, 0)),
            pl.BlockSpec((blk, 8), lambda i: (i, 0)),
        ],
        out_specs=pl.BlockSpec((blk, 512), lambda i: (i, 0)),
    )


def kernel(x, weight, scales):
    b, f = x.shape
    n = b * f
    nv = weight.shape[0]
    idx = x.reshape(n).astype(jnp.int32)
    w32 = lax.bitcast_convert_type(weight.reshape(nv, _W, 4), jnp.int32)
    scales_f32 = scales.astype(jnp.float32)
    rows_g, sc_g = _sc_gather(n)(idx, w32, scales_f32)
    out = _dequant(n, 512)(rows_g.reshape(n // 8, 128), sc_g.reshape(n // 8, 8))
    return out.astype(jnp.float16).reshape(b, f, _D)


# speedup vs baseline: 5.4642x; 1.3793x over previous
import functools

import jax
import jax.numpy as jnp
from jax import lax
from jax.experimental import pallas as pl
from jax.experimental.pallas import tpu as pltpu
from jax.experimental.pallas import tpu_sc as plsc

_D = 64
_W = _D // 4
_NW = 32
_CHUNK = 1024


_RCH = 1250


def _sc_repack_body(w_hbm, w32_out, src_v, dst_v, *, nv):
    rows_per_w = nv // _NW
    wid = lax.axis_index("s") * 2 + lax.axis_index("c")

    def chunk(c, _):
        row0 = wid * rows_per_w + c * _RCH
        pltpu.sync_copy(w_hbm.at[pl.ds(row0 * _D, _RCH * _D)], src_v)

        def body(i, _):
            for u in range(10):
                r = i * 10 + u
                dst_v[r, :] = plsc.bitcast(src_v[pl.ds(r * _D, _D)], jnp.int32)
            return 0

        lax.fori_loop(0, _RCH // 10, body, 0)
        pltpu.sync_copy(dst_v, w32_out.at[pl.ds(row0, _RCH)])
        return 0

    lax.fori_loop(0, rows_per_w // _RCH, chunk, 0)


@functools.cache
def _sc_repack(nv):
    mesh = plsc.VectorSubcoreMesh(core_axis_name="c", subcore_axis_name="s")
    return pl.kernel(
        functools.partial(_sc_repack_body, nv=nv),
        out_type=jax.ShapeDtypeStruct((nv, _W), jnp.int32),
        mesh=mesh,
        compiler_params=pltpu.CompilerParams(
            use_tc_tiling_on_sc=False, needs_layout_passes=False),
        scratch_types=[
            pltpu.VMEM((_RCH * _D,), jnp.int8),
            pltpu.VMEM((_RCH, _W), jnp.int32),
        ],
    )


def _sc_gather_body(idx_hbm, w_hbm, s_hbm, rows_out, sc_out,
                    idx_v, rows_v, sc_v, sem, *, n_per_w):
    w32_hbm = w_hbm
    wid = lax.axis_index("s") * 2 + lax.axis_index("c")
    for k in range(n_per_w // _CHUNK):
        base = wid * n_per_w + k * _CHUNK
        pltpu.sync_copy(idx_hbm.at[pl.ds(base, _CHUNK)], idx_v)
        rows_dma = pltpu.async_copy(w32_hbm.at[idx_v], rows_v, sem)
        sc_dma = pltpu.async_copy(s_hbm.at[idx_v], sc_v, sem)
        rows_dma.wait()
        sc_dma.wait()
        pltpu.sync_copy(rows_v, rows_out.at[pl.ds(base, _CHUNK)])
        pltpu.sync_copy(sc_v, sc_out.at[pl.ds(base, _CHUNK)])


@functools.cache
def _sc_gather(n):
    n_per_w = n // _NW
    mesh = plsc.VectorSubcoreMesh(core_axis_name="c", subcore_axis_name="s")
    return pl.kernel(
        functools.partial(_sc_gather_body, n_per_w=n_per_w),
        out_type=[
            jax.ShapeDtypeStruct((n, _W), jnp.int32),
            jax.ShapeDtypeStruct((n,), jnp.float32),
        ],
        mesh=mesh,
        compiler_params=pltpu.CompilerParams(use_tc_tiling_on_sc=False),
        scratch_types=[
            pltpu.VMEM((_CHUNK,), jnp.int32),
            pltpu.VMEM((_CHUNK, _W), jnp.int32),
            pltpu.VMEM((_CHUNK,), jnp.float32),
            pltpu.SemaphoreType.DMA,
        ],
    )


def _dequant_body(rows_ref, sc_ref, out_ref):
    w = rows_ref[...]
    s = sc_ref[...]
    blk = w.shape[0]
    lane = lax.broadcasted_iota(jnp.int32, (blk, 512), 1)
    wrep = jnp.take_along_axis(w, lane // 4, axis=1)
    srep = jnp.take_along_axis(s, lane // _D, axis=1)
    k = lax.broadcasted_iota(jnp.int32, (blk, 512), 1) % 4
    b = (wrep << (24 - 8 * k)) >> 24
    out_ref[...] = b.astype(jnp.float32) * srep


@functools.cache
def _dequant(n, blk):
    return pl.pallas_call(
        _dequant_body,
        out_shape=jax.ShapeDtypeStruct((n // 8, 512), jnp.float32),
        grid=(n // 8 // blk,),
        in_specs=[
            pl.BlockSpec((blk, 128), lambda i: (i, 0)),
            pl.BlockSpec((blk, 8), lambda i: (i, 0)),
        ],
        out_specs=pl.BlockSpec((blk, 512), lambda i: (i, 0)),
    )


def kernel(x, weight, scales):
    b, f = x.shape
    n = b * f
    nv = weight.shape[0]
    idx = x.reshape(n).astype(jnp.int32)
    scales_f32 = scales.astype(jnp.float32)
    w32 = _sc_repack(nv)(weight.reshape(nv * _D))
    rows_g, sc_g = _sc_gather(n)(idx, w32, scales_f32)
    out = _dequant(n, 512)(rows_g.reshape(n // 8, 128), sc_g.reshape(n // 8, 8))
    return out.astype(jnp.float16).reshape(b, f, _D)


# speedup vs baseline: 5.4660x; 1.0003x over previous
import functools

import jax
import jax.numpy as jnp
from jax import lax
from jax.experimental import pallas as pl
from jax.experimental.pallas import tpu as pltpu
from jax.experimental.pallas import tpu_sc as plsc

_D = 64
_W = _D // 4
_NW = 32
_CHUNK = 1024


_RCH = 1250


def _sc_repack_body(w_hbm, w32_out, src_v, dst_v, *, nv):
    rows_per_w = nv // _NW
    wid = lax.axis_index("s") * 2 + lax.axis_index("c")

    def chunk(c, _):
        row0 = wid * rows_per_w + c * _RCH
        pltpu.sync_copy(w_hbm.at[pl.ds(row0, _RCH)], src_v)

        def body(i, _):
            for u in range(10):
                r = i * 10 + u
                dst_v[r, :] = plsc.bitcast(src_v[r, :], jnp.int32)
            return 0

        lax.fori_loop(0, _RCH // 10, body, 0)
        pltpu.sync_copy(dst_v, w32_out.at[pl.ds(row0, _RCH)])
        return 0

    lax.fori_loop(0, rows_per_w // _RCH, chunk, 0)


@functools.cache
def _sc_repack(nv):
    mesh = plsc.VectorSubcoreMesh(core_axis_name="c", subcore_axis_name="s")
    return pl.kernel(
        functools.partial(_sc_repack_body, nv=nv),
        out_type=jax.ShapeDtypeStruct((nv, _W), jnp.int32),
        mesh=mesh,
        compiler_params=pltpu.CompilerParams(
            use_tc_tiling_on_sc=False, needs_layout_passes=False),
        scratch_types=[
            pltpu.VMEM((_RCH, _D), jnp.int8),
            pltpu.VMEM((_RCH, _W), jnp.int32),
        ],
    )


def _sc_gather_body(idx_hbm, w_hbm, s_hbm, rows_out, sc_out,
                    idx_v, rows_v, sc_v, sem, *, n_per_w):
    w32_hbm = w_hbm
    wid = lax.axis_index("s") * 2 + lax.axis_index("c")
    for k in range(n_per_w // _CHUNK):
        base = wid * n_per_w + k * _CHUNK
        pltpu.sync_copy(idx_hbm.at[pl.ds(base, _CHUNK)], idx_v)
        rows_dma = pltpu.async_copy(w32_hbm.at[idx_v], rows_v, sem)
        sc_dma = pltpu.async_copy(s_hbm.at[idx_v], sc_v, sem)
        rows_dma.wait()
        sc_dma.wait()
        pltpu.sync_copy(rows_v, rows_out.at[pl.ds(base, _CHUNK)])
        pltpu.sync_copy(sc_v, sc_out.at[pl.ds(base, _CHUNK)])


@functools.cache
def _sc_gather(n):
    n_per_w = n // _NW
    mesh = plsc.VectorSubcoreMesh(core_axis_name="c", subcore_axis_name="s")
    return pl.kernel(
        functools.partial(_sc_gather_body, n_per_w=n_per_w),
        out_type=[
            jax.ShapeDtypeStruct((n, _W), jnp.int32),
            jax.ShapeDtypeStruct((n,), jnp.float32),
        ],
        mesh=mesh,
        compiler_params=pltpu.CompilerParams(use_tc_tiling_on_sc=False),
        scratch_types=[
            pltpu.VMEM((_CHUNK,), jnp.int32),
            pltpu.VMEM((_CHUNK, _W), jnp.int32),
            pltpu.VMEM((_CHUNK,), jnp.float32),
            pltpu.SemaphoreType.DMA,
        ],
    )


def _dequant_body(rows_ref, sc_ref, out_ref):
    w = rows_ref[...]
    s = sc_ref[...]
    blk = w.shape[0]
    lane = lax.broadcasted_iota(jnp.int32, (blk, 512), 1)
    wrep = jnp.take_along_axis(w, lane // 4, axis=1)
    srep = jnp.take_along_axis(s, lane // _D, axis=1)
    k = lax.broadcasted_iota(jnp.int32, (blk, 512), 1) % 4
    b = (wrep << (24 - 8 * k)) >> 24
    out_ref[...] = b.astype(jnp.float32) * srep


@functools.cache
def _dequant(n, blk):
    return pl.pallas_call(
        _dequant_body,
        out_shape=jax.ShapeDtypeStruct((n // 8, 512), jnp.float32),
        grid=(n // 8 // blk,),
        in_specs=[
            pl.BlockSpec((blk, 128), lambda i: (i, 0)),
            pl.BlockSpec((blk, 8), lambda i: (i, 0)),
        ],
        out_specs=pl.BlockSpec((blk, 512), lambda i: (i, 0)),
    )


def kernel(x, weight, scales):
    b, f = x.shape
    n = b * f
    nv = weight.shape[0]
    idx = x.reshape(n).astype(jnp.int32)
    scales_f32 = scales.astype(jnp.float32)
    w32 = _sc_repack(nv)(weight)
    rows_g, sc_g = _sc_gather(n)(idx, w32, scales_f32)
    out = _dequant(n, 512)(rows_g.reshape(n // 8, 128), sc_g.reshape(n // 8, 8))
    return out.astype(jnp.float16).reshape(b, f, _D)
